# baseline (speedup 1.0000x reference)
import jax
import jax.numpy as jnp
from jax import lax
from jax.experimental import pallas as pl
from jax.experimental.pallas import tpu as pltpu

N_ROWS = 100000
TILE = 4000


def _mlp_kernel(rx_ref, tx_ref,
                W_enc_ref, b_enc_ref,
                Wy1_ref, by1_ref, Wy2_ref, by2_ref,
                Wac1_ref, bac1_ref, Wac2_ref, bac2_ref,
                outy_ref, outac_ref,
                MyT_ref, cy_ref, MacT_ref, cac_ref):
    i = pl.program_id(0)

    @pl.when(i == 0)
    def _fold_weights():
        MyT_ref[...] = lax.dot_general(
            W_enc_ref[...], Wy1_ref[...], (((0,), (1,)), ((), ())),
            preferred_element_type=jnp.float32)
        cy_ref[...] = lax.dot_general(
            b_enc_ref[...], Wy1_ref[...], (((1,), (1,)), ((), ())),
            preferred_element_type=jnp.float32) + by1_ref[...]
        MacT_ref[...] = lax.dot_general(
            W_enc_ref[...], Wac1_ref[...], (((0,), (1,)), ((), ())),
            preferred_element_type=jnp.float32)
        cac_ref[...] = lax.dot_general(
            b_enc_ref[...], Wac1_ref[...], (((1,), (1,)), ((), ())),
            preferred_element_type=jnp.float32) + bac1_ref[...]

    hy = jnp.maximum(
        jnp.dot(rx_ref[...], MyT_ref[...],
                preferred_element_type=jnp.float32) + cy_ref[...], 0.0)
    outy_ref[...] = jnp.sum(hy * Wy2_ref[...], axis=1,
                            keepdims=True) + by2_ref[...]

    hac = jnp.maximum(
        jnp.dot(tx_ref[...], MacT_ref[...],
                preferred_element_type=jnp.float32) + cac_ref[...], 0.0)
    outac_ref[...] = jnp.sum(hac * Wac2_ref[...], axis=1,
                             keepdims=True) + bac2_ref[...]


def kernel(mol_x, reaction_x, target_x, W_enc, b_enc, Wa1, ba1, Wa2, ba2,
           W_upd, b_upd, Wy1, by1, Wy2, by2, Wac1, bac1, Wac2, bac2):
    del mol_x, Wa1, ba1, Wa2, ba2, W_upd, b_upd
    n = reaction_x.shape[0]
    grid = (n // TILE,)

    b_enc2 = b_enc.reshape(1, -1)
    by1_2 = by1.reshape(1, -1)
    by2_2 = by2.reshape(1, 1)
    bac1_2 = bac1.reshape(1, -1)
    bac2_2 = bac2.reshape(1, 1)

    row_spec = pl.BlockSpec((TILE, 128), lambda i: (i, 0))
    out_spec = pl.BlockSpec((TILE, 1), lambda i: (i, 0))

    def whole(shape):
        return pl.BlockSpec(shape, lambda i: tuple(0 for _ in shape))

    outy, outac = pl.pallas_call(
        _mlp_kernel,
        grid=grid,
        in_specs=[
            row_spec, row_spec,
            whole((128, 128)), whole((1, 128)),
            whole((64, 128)), whole((1, 64)), whole((1, 64)), whole((1, 1)),
            whole((64, 128)), whole((1, 64)), whole((1, 64)), whole((1, 1)),
        ],
        out_specs=[out_spec, out_spec],
        out_shape=[
            jax.ShapeDtypeStruct((n, 1), jnp.float32),
            jax.ShapeDtypeStruct((n, 1), jnp.float32),
        ],
        scratch_shapes=[
            pltpu.VMEM((128, 64), jnp.float32),
            pltpu.VMEM((1, 64), jnp.float32),
            pltpu.VMEM((128, 64), jnp.float32),
            pltpu.VMEM((1, 64), jnp.float32),
        ],
        compiler_params=pltpu.CompilerParams(
            dimension_semantics=("arbitrary",)),
    )(reaction_x, target_x,
      W_enc, b_enc2,
      Wy1, by1_2, Wy2, by2_2,
      Wac1, bac1_2, Wac2, bac2_2)

    return (outy[:, 0], outac[:, 0])


# speedup vs baseline: 1.8657x; 1.8657x over previous
import jax
import jax.numpy as jnp
from jax import lax
from jax.experimental import pallas as pl
from jax.experimental.pallas import tpu as pltpu

N_ROWS = 100000
TILE = 4000


def _mlp_kernel(rx_ref, tx_ref,
                W_enc_ref, b_enc_ref,
                Wy1_ref, by1_ref, Wy2_ref, by2_ref,
                Wac1_ref, bac1_ref, Wac2_ref, bac2_ref,
                outy_ref, outac_ref,
                MyT_ref, cy_ref, MacT_ref, cac_ref):
    i = pl.program_id(0)

    @pl.when(i == 0)
    def _fold_weights():
        MyT_ref[...] = lax.dot_general(
            W_enc_ref[...], Wy1_ref[...], (((0,), (1,)), ((), ())),
            preferred_element_type=jnp.float32)
        cy_ref[...] = lax.dot_general(
            b_enc_ref[...], Wy1_ref[...], (((1,), (1,)), ((), ())),
            preferred_element_type=jnp.float32) + by1_ref[...]
        MacT_ref[...] = lax.dot_general(
            W_enc_ref[...], Wac1_ref[...], (((0,), (1,)), ((), ())),
            preferred_element_type=jnp.float32)
        cac_ref[...] = lax.dot_general(
            b_enc_ref[...], Wac1_ref[...], (((1,), (1,)), ((), ())),
            preferred_element_type=jnp.float32) + bac1_ref[...]

    hy = jnp.maximum(
        jnp.dot(rx_ref[...], MyT_ref[...],
                preferred_element_type=jnp.float32) + cy_ref[...], 0.0)
    outy_ref[0] = lax.dot_general(
        Wy2_ref[...], hy, (((1,), (1,)), ((), ())),
        preferred_element_type=jnp.float32) + by2_ref[...]

    hac = jnp.maximum(
        jnp.dot(tx_ref[...], MacT_ref[...],
                preferred_element_type=jnp.float32) + cac_ref[...], 0.0)
    outac_ref[0] = lax.dot_general(
        Wac2_ref[...], hac, (((1,), (1,)), ((), ())),
        preferred_element_type=jnp.float32) + bac2_ref[...]


def kernel(mol_x, reaction_x, target_x, W_enc, b_enc, Wa1, ba1, Wa2, ba2,
           W_upd, b_upd, Wy1, by1, Wy2, by2, Wac1, bac1, Wac2, bac2):
    del mol_x, Wa1, ba1, Wa2, ba2, W_upd, b_upd
    n = reaction_x.shape[0]
    grid = (n // TILE,)

    b_enc2 = b_enc.reshape(1, -1)
    by1_2 = by1.reshape(1, -1)
    by2_2 = by2.reshape(1, 1)
    bac1_2 = bac1.reshape(1, -1)
    bac2_2 = bac2.reshape(1, 1)

    row_spec = pl.BlockSpec((TILE, 128), lambda i: (i, 0))
    out_spec = pl.BlockSpec((1, 1, TILE), lambda i: (i, 0, 0))

    def whole(shape):
        return pl.BlockSpec(shape, lambda i: tuple(0 for _ in shape))

    outy, outac = pl.pallas_call(
        _mlp_kernel,
        grid=grid,
        in_specs=[
            row_spec, row_spec,
            whole((128, 128)), whole((1, 128)),
            whole((64, 128)), whole((1, 64)), whole((1, 64)), whole((1, 1)),
            whole((64, 128)), whole((1, 64)), whole((1, 64)), whole((1, 1)),
        ],
        out_specs=[out_spec, out_spec],
        out_shape=[
            jax.ShapeDtypeStruct((n // TILE, 1, TILE), jnp.float32),
            jax.ShapeDtypeStruct((n // TILE, 1, TILE), jnp.float32),
        ],
        scratch_shapes=[
            pltpu.VMEM((128, 64), jnp.float32),
            pltpu.VMEM((1, 64), jnp.float32),
            pltpu.VMEM((128, 64), jnp.float32),
            pltpu.VMEM((1, 64), jnp.float32),
        ],
        compiler_params=pltpu.CompilerParams(
            dimension_semantics=("arbitrary",)),
    )(reaction_x, target_x,
      W_enc, b_enc2,
      Wy1, by1_2, Wy2, by2_2,
      Wac1, bac1_2, Wac2, bac2_2)

    return (outy.reshape(-1), outac.reshape(-1))


# speedup vs baseline: 2.2767x; 1.2203x over previous
import jax
import jax.numpy as jnp
from jax import lax
from jax.experimental import pallas as pl
from jax.experimental.pallas import tpu as pltpu

N_ROWS = 100000
TILE = 10000


def _mlp_kernel(rx_ref, tx_ref,
                W_enc_ref, b_enc_ref,
                Wy1_ref, by1_ref, Wy2_ref, by2_ref,
                Wac1_ref, bac1_ref, Wac2_ref, bac2_ref,
                outy_ref, outac_ref,
                MyT_ref, cy_ref, MacT_ref, cac_ref):
    i = pl.program_id(0)

    @pl.when(i == 0)
    def _fold_weights():
        MyT_ref[...] = lax.dot_general(
            W_enc_ref[...], Wy1_ref[...], (((0,), (1,)), ((), ())),
            preferred_element_type=jnp.float32)
        cy_ref[...] = lax.dot_general(
            b_enc_ref[...], Wy1_ref[...], (((1,), (1,)), ((), ())),
            preferred_element_type=jnp.float32) + by1_ref[...]
        MacT_ref[...] = lax.dot_general(
            W_enc_ref[...], Wac1_ref[...], (((0,), (1,)), ((), ())),
            preferred_element_type=jnp.float32)
        cac_ref[...] = lax.dot_general(
            b_enc_ref[...], Wac1_ref[...], (((1,), (1,)), ((), ())),
            preferred_element_type=jnp.float32) + bac1_ref[...]

    hy = jnp.maximum(
        jnp.dot(rx_ref[...], MyT_ref[...],
                preferred_element_type=jnp.float32) + cy_ref[...], 0.0)
    outy_ref[0] = lax.dot_general(
        Wy2_ref[...], hy, (((1,), (1,)), ((), ())),
        preferred_element_type=jnp.float32) + by2_ref[...]

    hac = jnp.maximum(
        jnp.dot(tx_ref[...], MacT_ref[...],
                preferred_element_type=jnp.float32) + cac_ref[...], 0.0)
    outac_ref[0] = lax.dot_general(
        Wac2_ref[...], hac, (((1,), (1,)), ((), ())),
        preferred_element_type=jnp.float32) + bac2_ref[...]


def kernel(mol_x, reaction_x, target_x, W_enc, b_enc, Wa1, ba1, Wa2, ba2,
           W_upd, b_upd, Wy1, by1, Wy2, by2, Wac1, bac1, Wac2, bac2):
    del mol_x, Wa1, ba1, Wa2, ba2, W_upd, b_upd
    n = reaction_x.shape[0]
    grid = (n // TILE,)

    b_enc2 = b_enc.reshape(1, -1)
    by1_2 = by1.reshape(1, -1)
    by2_2 = by2.reshape(1, 1)
    bac1_2 = bac1.reshape(1, -1)
    bac2_2 = bac2.reshape(1, 1)

    row_spec = pl.BlockSpec((TILE, 128), lambda i: (i, 0))
    out_spec = pl.BlockSpec((1, 1, TILE), lambda i: (i, 0, 0))

    def whole(shape):
        return pl.BlockSpec(shape, lambda i: tuple(0 for _ in shape))

    outy, outac = pl.pallas_call(
        _mlp_kernel,
        grid=grid,
        in_specs=[
            row_spec, row_spec,
            whole((128, 128)), whole((1, 128)),
            whole((64, 128)), whole((1, 64)), whole((1, 64)), whole((1, 1)),
            whole((64, 128)), whole((1, 64)), whole((1, 64)), whole((1, 1)),
        ],
        out_specs=[out_spec, out_spec],
        out_shape=[
            jax.ShapeDtypeStruct((n // TILE, 1, TILE), jnp.float32),
            jax.ShapeDtypeStruct((n // TILE, 1, TILE), jnp.float32),
        ],
        scratch_shapes=[
            pltpu.VMEM((128, 64), jnp.float32),
            pltpu.VMEM((1, 64), jnp.float32),
            pltpu.VMEM((128, 64), jnp.float32),
            pltpu.VMEM((1, 64), jnp.float32),
        ],
        compiler_params=pltpu.CompilerParams(
            dimension_semantics=("arbitrary",)),
    )(reaction_x, target_x,
      W_enc, b_enc2,
      Wy1, by1_2, Wy2, by2_2,
      Wac1, bac1_2, Wac2, bac2_2)

    return (outy.reshape(-1), outac.reshape(-1))
